# baseline (device time: 14119 ns/iter reference)
import jax
import jax.numpy as jnp
from jax import lax
from jax.experimental import pallas as pl
from jax.experimental.pallas import tpu as pltpu

CH = 16
J = 32


def kernel(x):
    m, n = x.shape
    half = m // 2
    fwd = half - J
    F = fwd // CH
    assert F * CH == fwd and J % 8 == 0

    def body(x_ref, out_ref, raw_ref, send_x, recv_x, send_y, recv_y):
        my_x = lax.axis_index("x")
        my_y = lax.axis_index("y")
        x_peer = (1 - my_x, my_y)
        y_peer = (my_x, 1 - my_y)

        barrier_sem = pltpu.get_barrier_semaphore()
        for peer in (x_peer, y_peer):
            pl.semaphore_signal(
                barrier_sem, inc=1, device_id=peer,
                device_id_type=pl.DeviceIdType.MESH,
            )
        pl.semaphore_wait(barrier_sem, 2)

        def run(base, ext):
            def start_x(src_row, dst_row, rows, c):
                rd = pltpu.make_async_remote_copy(
                    src_ref=x_ref.at[pl.ds(src_row, rows), :],
                    dst_ref=raw_ref.at[pl.ds(dst_row, rows), :],
                    send_sem=send_x.at[c],
                    recv_sem=recv_x.at[c],
                    device_id=x_peer,
                    device_id_type=pl.DeviceIdType.MESH,
                )
                rd.start()
                return rd

            x_rdmas = [
                start_x(base + c * CH, c * CH, CH, c) for c in range(F)
            ]
            own_tail = start_x(base + fwd, fwd, J, F)
            extension = start_x(ext, fwd + J, J, F + 1)

            y_rdmas = []
            for c in range(F):
                x_rdmas[c].wait_recv()
                out_ref[pl.ds(base + c * CH, CH), :] = (
                    x_ref[pl.ds(base + c * CH, CH), :]
                    + raw_ref[pl.ds(c * CH, CH), :]
                )
                rd = pltpu.make_async_remote_copy(
                    src_ref=out_ref.at[pl.ds(base + c * CH, CH), :],
                    dst_ref=out_ref.at[pl.ds(base + c * CH, CH), :],
                    send_sem=send_y.at[c],
                    recv_sem=recv_y.at[c],
                    device_id=y_peer,
                    device_id_type=pl.DeviceIdType.MESH,
                )
                rd.start()
                y_rdmas.append(rd)

            own_tail.wait_recv()
            out_ref[pl.ds(base + fwd, J), :] = (
                x_ref[pl.ds(base + fwd, J), :]
                + raw_ref[pl.ds(fwd, J), :]
            )
            extension.wait_recv()
            out_ref[pl.ds(ext, J), :] = (
                x_ref[pl.ds(ext, J), :] + raw_ref[pl.ds(fwd + J, J), :]
            )

            for c in range(F):
                y_rdmas[c].wait_recv()
            for rd in x_rdmas + [own_tail, extension] + y_rdmas:
                rd.wait_send()

        @pl.when(my_y == 0)
        def _():
            run(0, m - J)

        @pl.when(my_y == 1)
        def _():
            run(half, half - J)

    return pl.pallas_call(
        body,
        out_shape=jax.ShapeDtypeStruct((m, n), x.dtype),
        in_specs=[pl.BlockSpec(memory_space=pltpu.VMEM)],
        out_specs=pl.BlockSpec(memory_space=pltpu.VMEM),
        scratch_shapes=[
            pltpu.VMEM((half + J, n), x.dtype),
            pltpu.SemaphoreType.DMA((F + 2,)),
            pltpu.SemaphoreType.DMA((F + 2,)),
            pltpu.SemaphoreType.DMA((F,)),
            pltpu.SemaphoreType.DMA((F,)),
        ],
        compiler_params=pltpu.CompilerParams(collective_id=0),
    )(x)
